# baseline (device time: 114728 ns/iter reference)
import jax
import jax.numpy as jnp
from jax import lax
from jax.experimental import pallas as pl
from jax.experimental.pallas import tpu as pltpu

N_DEV = 8
N_LAYERS = 3
_XOR_MASKS = (1, 3, 4)
NT = 8


def kernel(x, Win0, Wout0, Win1, Wout1, Win2, Wout2):
    b, d = x.shape
    d_in, h_per = Win0.shape
    t = h_per // NT

    def body(x_ref, w0i_ref, w0o_ref, w1i_ref, w1o_ref, w2i_ref, w2o_ref,
             out_ref, xbf_ref, acc_ref, comm_ref, recv_ref,
             send_sems, recv_sems):
        l = pl.program_id(0)
        j = pl.program_id(1)
        my = lax.axis_index("i")

        @pl.when((l == 0) & (j == 0))
        def _entry():
            barrier = pltpu.get_barrier_semaphore()
            for m in _XOR_MASKS:
                pl.semaphore_signal(
                    barrier, inc=1,
                    device_id=(my ^ m,),
                    device_id_type=pl.DeviceIdType.MESH,
                )
            pl.semaphore_wait(barrier, len(_XOR_MASKS))
            xbf_ref[...] = x_ref[...].astype(jnp.bfloat16)

        @pl.when(j == 0)
        def _zero():
            acc_ref[...] = jnp.zeros((b, d), jnp.float32)

        for k, (wi_ref, wo_ref) in enumerate(
            ((w0i_ref, w0o_ref), (w1i_ref, w1o_ref), (w2i_ref, w2o_ref))
        ):
            @pl.when(l == k)
            def _compute(wi_ref=wi_ref, wo_ref=wo_ref):
                wi = wi_ref[...].astype(jnp.bfloat16)
                h = lax.dot_general(
                    xbf_ref[...], wi, (((1,), (0,)), ((), ())),
                    preferred_element_type=jnp.float32,
                )
                h = jnp.maximum(h, 0.0).astype(jnp.bfloat16)
                wo = wo_ref[...].astype(jnp.bfloat16)
                acc_ref[...] += lax.dot_general(
                    h, wo, (((1,), (0,)), ((), ())),
                    preferred_element_type=jnp.float32,
                )

        @pl.when(j == NT - 1)
        def _allreduce():
            comm_ref[0, :, :] = acc_ref[...].astype(jnp.bfloat16)
            for k in range(N_LAYERS):
                @pl.when(l == k)
                def _rounds(k=k):
                    for r, m in enumerate(_XOR_MASKS):
                        s = N_LAYERS * k + r
                        rdma = pltpu.make_async_remote_copy(
                            src_ref=comm_ref.at[r],
                            dst_ref=recv_ref.at[r],
                            send_sem=send_sems.at[s],
                            recv_sem=recv_sems.at[s],
                            device_id=(my ^ m,),
                            device_id_type=pl.DeviceIdType.MESH,
                        )
                        rdma.start()
                        rdma.wait()
                        comm_ref[r + 1, :, :] = (
                            comm_ref[r].astype(jnp.float32)
                            + recv_ref[r].astype(jnp.float32)
                        ).astype(jnp.bfloat16)
            xbf_ref[...] = comm_ref[len(_XOR_MASKS)]

            @pl.when(l == N_LAYERS - 1)
            def _store():
                out_ref[...] = comm_ref[len(_XOR_MASKS)].astype(jnp.float32)

    def win_map(k):
        return lambda l, j: (
            0, jnp.where(l < k, 0, jnp.where(l > k, NT - 1, j))
        )

    def wout_map(k):
        return lambda l, j: (
            jnp.where(l < k, 0, jnp.where(l > k, NT - 1, j)), 0
        )

    return pl.pallas_call(
        body,
        grid=(N_LAYERS, NT),
        out_shape=jax.ShapeDtypeStruct((b, d), jnp.float32),
        in_specs=[
            pl.BlockSpec((b, d), lambda l, j: (0, 0)),
            pl.BlockSpec((d_in, t), win_map(0)),
            pl.BlockSpec((t, d), wout_map(0)),
            pl.BlockSpec((d_in, t), win_map(1)),
            pl.BlockSpec((t, d), wout_map(1)),
            pl.BlockSpec((d_in, t), win_map(2)),
            pl.BlockSpec((t, d), wout_map(2)),
        ],
        out_specs=pl.BlockSpec((b, d), lambda l, j: (0, 0)),
        scratch_shapes=[
            pltpu.VMEM((b, d), jnp.bfloat16),
            pltpu.VMEM((b, d), jnp.float32),
            pltpu.VMEM((4, b, d), jnp.bfloat16),
            pltpu.VMEM((3, b, d), jnp.bfloat16),
            pltpu.SemaphoreType.DMA((N_LAYERS * 3,)),
            pltpu.SemaphoreType.DMA((N_LAYERS * 3,)),
        ],
        compiler_params=pltpu.CompilerParams(
            dimension_semantics=("arbitrary", "arbitrary"),
            collective_id=0,
            vmem_limit_bytes=60 * 1024 * 1024,
        ),
    )(x, Win0, Wout0, Win1, Wout1, Win2, Wout2)


# device time: 74322 ns/iter; 1.5437x vs baseline; 1.5437x over previous
import os

import jax
import jax.numpy as jnp
from jax import lax
from jax.experimental import pallas as pl
from jax.experimental.pallas import tpu as pltpu

_EXPT = os.environ.get("EXPT", "full")

N_DEV = 8
N_LAYERS = 3
_XOR_MASKS = (1, 3, 4)
NT = 8


def kernel(x, Win0, Wout0, Win1, Wout1, Win2, Wout2):
    b, d = x.shape
    d_in, h_per = Win0.shape
    t = h_per // NT

    def body(x_ref, w0i_ref, w0o_ref, w1i_ref, w1o_ref, w2i_ref, w2o_ref,
             out_ref, xbf_ref, acc_ref, comm_ref, recv_ref,
             send_sems, recv_sems):
        l = pl.program_id(0)
        j = pl.program_id(1)
        my = lax.axis_index("i")

        @pl.when((l == 0) & (j == 0))
        def _entry():
            barrier = pltpu.get_barrier_semaphore()
            for m in _XOR_MASKS:
                pl.semaphore_signal(
                    barrier, inc=1,
                    device_id=(my ^ m,),
                    device_id_type=pl.DeviceIdType.MESH,
                )
            pl.semaphore_wait(barrier, len(_XOR_MASKS))
            xbf_ref[...] = x_ref[...].astype(jnp.bfloat16)

        @pl.when(j == 0)
        def _zero():
            acc_ref[...] = jnp.zeros((b, d), jnp.float32)

        for k, (wi_ref, wo_ref) in enumerate(
            ((w0i_ref, w0o_ref), (w1i_ref, w1o_ref), (w2i_ref, w2o_ref))
        ):
            if _EXPT == "stream":
                break

            @pl.when(l == k)
            def _compute(wi_ref=wi_ref, wo_ref=wo_ref):
                wi = wi_ref[...].astype(jnp.bfloat16)
                h = lax.dot_general(
                    xbf_ref[...], wi, (((1,), (0,)), ((), ())),
                    preferred_element_type=jnp.float32,
                )
                h = jnp.maximum(h, 0.0).astype(jnp.bfloat16)
                wo = wo_ref[...].astype(jnp.bfloat16)
                acc_ref[...] += lax.dot_general(
                    h, wo, (((1,), (0,)), ((), ())),
                    preferred_element_type=jnp.float32,
                )

        @pl.when(j == NT - 1)
        def _allreduce():
            comm_ref[0, :, :] = acc_ref[...].astype(jnp.bfloat16)
            rounds = range(0) if _EXPT in ("stream", "nordma") else range(N_LAYERS)
            for k in rounds:
                @pl.when(l == k)
                def _rounds(k=k):
                    for r, m in enumerate(_XOR_MASKS):
                        s = N_LAYERS * k + r
                        rdma = pltpu.make_async_remote_copy(
                            src_ref=comm_ref.at[r],
                            dst_ref=recv_ref.at[r],
                            send_sem=send_sems.at[s],
                            recv_sem=recv_sems.at[s],
                            device_id=(my ^ m,),
                            device_id_type=pl.DeviceIdType.MESH,
                        )
                        rdma.start()
                        rdma.wait()
                        comm_ref[r + 1, :, :] = (
                            comm_ref[r].astype(jnp.float32)
                            + recv_ref[r].astype(jnp.float32)
                        ).astype(jnp.bfloat16)
            xbf_ref[...] = comm_ref[len(_XOR_MASKS)]

            @pl.when(l == N_LAYERS - 1)
            def _store():
                out_ref[...] = comm_ref[len(_XOR_MASKS)].astype(jnp.float32)

    def win_map(k):
        return lambda l, j: (
            0, jnp.where(l < k, 0, jnp.where(l > k, NT - 1, j))
        )

    def wout_map(k):
        return lambda l, j: (
            jnp.where(l < k, 0, jnp.where(l > k, NT - 1, j)), 0
        )

    return pl.pallas_call(
        body,
        grid=(N_LAYERS, NT),
        out_shape=jax.ShapeDtypeStruct((b, d), jnp.float32),
        in_specs=[
            pl.BlockSpec((b, d), lambda l, j: (0, 0)),
            pl.BlockSpec((d_in, t), win_map(0)),
            pl.BlockSpec((t, d), wout_map(0)),
            pl.BlockSpec((d_in, t), win_map(1)),
            pl.BlockSpec((t, d), wout_map(1)),
            pl.BlockSpec((d_in, t), win_map(2)),
            pl.BlockSpec((t, d), wout_map(2)),
        ],
        out_specs=pl.BlockSpec((b, d), lambda l, j: (0, 0)),
        scratch_shapes=[
            pltpu.VMEM((b, d), jnp.bfloat16),
            pltpu.VMEM((b, d), jnp.float32),
            pltpu.VMEM((4, b, d), jnp.bfloat16),
            pltpu.VMEM((3, b, d), jnp.bfloat16),
            pltpu.SemaphoreType.DMA((N_LAYERS * 3,)),
            pltpu.SemaphoreType.DMA((N_LAYERS * 3,)),
        ],
        compiler_params=pltpu.CompilerParams(
            dimension_semantics=("arbitrary", "arbitrary"),
            collective_id=0,
            vmem_limit_bytes=60 * 1024 * 1024,
        ),
    )(x, Win0, Wout0, Win1, Wout1, Win2, Wout2)
